# baseline (device time: 445567 ns/iter reference)
import jax
import jax.numpy as jnp
from jax import lax
from jax.experimental import pallas as pl
from jax.experimental.pallas import tpu as pltpu

NC = 8
CN = 512
BK = 256


def kernel(x, dy):
    K, Mx = x.shape
    H = Mx // 2
    N = dy.shape[1] // 2
    NK = K // BK
    assert N == NC * CN

    my_x_outer = lax.axis_index("x")
    s = jnp.stack([my_x_outer]).astype(jnp.int32)
    x = x.astype(jnp.bfloat16)
    dy = dy.astype(jnp.bfloat16)

    def body(s_ref, x_ref, dy_ref, out_ref, stage_ref, stage2_ref,
             pk, ps, o, psb, rvb, obf, cvb, cvo,
             ysend, yrecv, xsend, xrecv, lsem, lsem2):
        del s_ref
        c = pl.program_id(0)
        k = pl.program_id(1)
        my_x = lax.axis_index("x")
        my_y = lax.axis_index("y")
        y_tgt = (my_x, 1 - my_y)
        x_tgt = (1 - my_x, my_y)
        slot = lax.rem(c, 2)

        def y_rdma(cc, sl):
            return pltpu.make_async_remote_copy(
                src_ref=psb.at[sl],
                dst_ref=stage_ref.at[cc],
                send_sem=ysend.at[cc],
                recv_sem=yrecv.at[cc],
                device_id=y_tgt,
                device_id_type=pl.DeviceIdType.MESH,
            )

        def x_rdma(cc, sl):
            return pltpu.make_async_remote_copy(
                src_ref=obf.at[sl],
                dst_ref=stage2_ref.at[cc],
                send_sem=xsend.at[cc],
                recv_sem=xrecv.at[cc],
                device_id=x_tgt,
                device_id_type=pl.DeviceIdType.MESH,
            )

        def out_copy(cc, sl):
            return pltpu.make_async_copy(
                o.at[sl],
                out_ref.at[:, pl.ds(my_x * N + cc * CN, CN)],
                lsem.at[sl],
            )

        def cvt_copy(cc, sl):
            return pltpu.make_async_copy(
                cvo.at[sl],
                out_ref.at[:, pl.ds((1 - my_x) * N + cc * CN, CN)],
                lsem2.at[sl],
            )

        @pl.when((c == 0) & (k == 0))
        def _barrier():
            bar = pltpu.get_barrier_semaphore()
            for tgt in (y_tgt, x_tgt):
                pl.semaphore_signal(
                    bar, inc=1, device_id=tgt,
                    device_id_type=pl.DeviceIdType.MESH,
                )
            pl.semaphore_wait(bar, 2)

        @pl.when((k == 0) & (c >= 2))
        def _wait_prev_y():
            y_rdma(c - 2, slot).wait_send()

        @pl.when((k == 0) & (c >= 3))
        def _wait_prev_x():
            x_rdma(c - 3, 1 - slot).wait_send()
            out_copy(c - 3, 1 - slot).wait()

        @pl.when((k == 0) & (c >= 5))
        def _cvt_wait():
            cvt_copy(c - 5, 1 - slot).wait()

        @pl.when((k == 0) & (c >= 3))
        def _cvt():
            cc = c - 3
            sl = 1 - slot
            x_rdma(cc, sl).wait_recv()
            ld = pltpu.make_async_copy(
                stage2_ref.at[cc], cvb.at[sl], lsem2.at[sl]
            )
            ld.start()
            ld.wait()
            cvo[sl] = cvb[sl].astype(jnp.float32)
            cvt_copy(cc, sl).start()

        @pl.when(k == 0)
        def _zero():
            pk[slot] = jnp.zeros((H, CN), jnp.float32)
            ps[slot] = jnp.zeros((H, CN), jnp.float32)

        bb = dy_ref[...]
        a_keep = x_ref[:, pl.ds(my_y * H, H)]
        a_send = x_ref[:, pl.ds((1 - my_y) * H, H)]
        dn = (((0,), (0,)), ((), ()))
        pk[slot] += lax.dot_general(
            a_keep, bb, dn, preferred_element_type=jnp.float32
        )
        ps[slot] += lax.dot_general(
            a_send, bb, dn, preferred_element_type=jnp.float32
        )

        @pl.when(k == NK - 1)
        def _send_y():
            psb[slot] = ps[slot].astype(jnp.bfloat16)
            y_rdma(c, slot).start()

        def add_phase(cm1):
            sl = lax.rem(cm1, 2)
            y_rdma(cm1, sl).wait_recv()
            ld = pltpu.make_async_copy(
                stage_ref.at[cm1], rvb.at[sl], lsem.at[sl]
            )
            ld.start()
            ld.wait()
            ov = pk[sl] + rvb[sl].astype(jnp.float32)
            o[sl] = ov
            obf[sl] = ov.astype(jnp.bfloat16)
            out_copy(cm1, sl).start()
            x_rdma(cm1, sl).start()

        @pl.when((k == NK - 1) & (c >= 1))
        def _add_mid():
            add_phase(c - 1)

        @pl.when((c == NC - 1) & (k == NK - 1))
        def _final():
            sl_last = (NC - 1) % 2
            x_rdma(NC - 3, sl_last).wait_send()
            out_copy(NC - 3, sl_last).wait()
            add_phase(NC - 1)
            for cc in (NC - 3, NC - 2, NC - 1):
                sl = cc % 2
                cvt_copy(cc - 2, sl).wait()
                x_rdma(cc, sl).wait_recv()
                ld = pltpu.make_async_copy(
                    stage2_ref.at[cc], cvb.at[sl], lsem2.at[sl]
                )
                ld.start()
                ld.wait()
                cvo[sl] = cvb[sl].astype(jnp.float32)
                cvt_copy(cc, sl).start()
            for cc in (NC - 2, NC - 1):
                cvt_copy(cc, cc % 2).wait()
            for cc in (NC - 2, NC - 1):
                y_rdma(cc, cc % 2).wait_send()
                x_rdma(cc, cc % 2).wait_send()
                out_copy(cc, cc % 2).wait()

    grid_spec = pltpu.PrefetchScalarGridSpec(
        num_scalar_prefetch=1,
        grid=(NC, NK),
        in_specs=[
            pl.BlockSpec((BK, Mx), lambda c, k, s: (k, 0)),
            pl.BlockSpec((BK, CN), lambda c, k, s: (k, s[0] * NC + c)),
        ],
        out_specs=(
            pl.BlockSpec(memory_space=pltpu.MemorySpace.HBM),
            pl.BlockSpec(memory_space=pltpu.MemorySpace.HBM),
            pl.BlockSpec(memory_space=pltpu.MemorySpace.HBM),
        ),
        scratch_shapes=[
            pltpu.VMEM((2, H, CN), jnp.float32),
            pltpu.VMEM((2, H, CN), jnp.float32),
            pltpu.VMEM((2, H, CN), jnp.float32),
            pltpu.VMEM((2, H, CN), jnp.bfloat16),
            pltpu.VMEM((2, H, CN), jnp.bfloat16),
            pltpu.VMEM((2, H, CN), jnp.bfloat16),
            pltpu.VMEM((2, H, CN), jnp.bfloat16),
            pltpu.VMEM((2, H, CN), jnp.float32),
            pltpu.SemaphoreType.DMA((NC,)),
            pltpu.SemaphoreType.DMA((NC,)),
            pltpu.SemaphoreType.DMA((NC,)),
            pltpu.SemaphoreType.DMA((NC,)),
            pltpu.SemaphoreType.DMA((2,)),
            pltpu.SemaphoreType.DMA((2,)),
        ],
    )
    out, _, _ = pl.pallas_call(
        body,
        grid_spec=grid_spec,
        out_shape=(
            jax.ShapeDtypeStruct((H, 2 * N), jnp.float32),
            jax.ShapeDtypeStruct((NC, H, CN), jnp.bfloat16),
            jax.ShapeDtypeStruct((NC, H, CN), jnp.bfloat16),
        ),
        compiler_params=pltpu.CompilerParams(
            dimension_semantics=("arbitrary", "arbitrary"),
            collective_id=0,
            has_side_effects=True,
            vmem_limit_bytes=62 * 1024 * 1024,
        ),
    )(s, x, dy)
    return out


# device time: 398161 ns/iter; 1.1191x vs baseline; 1.1191x over previous
import jax
import jax.numpy as jnp
from jax import lax
from jax.experimental import pallas as pl
from jax.experimental.pallas import tpu as pltpu

NC = 8
CN = 512
BK = 256


def kernel(x, dy):
    K, Mx = x.shape
    H = Mx // 2
    N = dy.shape[1] // 2
    NK = K // BK
    assert N == NC * CN

    my_x_outer = lax.axis_index("x")
    s = jnp.stack([my_x_outer]).astype(jnp.int32)

    def body(s_ref, x_ref, dy_ref, out_ref, stage_ref, stage2_ref,
             acc, o, psb, rvb, obf, cvb, cvo,
             ysend, yrecv, xsend, xrecv, lsem, lsem2):
        del s_ref
        c = pl.program_id(0)
        k = pl.program_id(1)
        my_x = lax.axis_index("x")
        my_y = lax.axis_index("y")
        y_tgt = (my_x, 1 - my_y)
        x_tgt = (1 - my_x, my_y)
        slot = lax.rem(c, 2)

        def y_rdma(cc, sl):
            return pltpu.make_async_remote_copy(
                src_ref=psb.at[sl],
                dst_ref=stage_ref.at[cc],
                send_sem=ysend.at[cc],
                recv_sem=yrecv.at[cc],
                device_id=y_tgt,
                device_id_type=pl.DeviceIdType.MESH,
            )

        def x_rdma(cc, sl):
            return pltpu.make_async_remote_copy(
                src_ref=obf.at[sl],
                dst_ref=stage2_ref.at[cc],
                send_sem=xsend.at[cc],
                recv_sem=xrecv.at[cc],
                device_id=x_tgt,
                device_id_type=pl.DeviceIdType.MESH,
            )

        def out_copy(cc, sl):
            return pltpu.make_async_copy(
                o.at[sl],
                out_ref.at[:, pl.ds(my_x * N + cc * CN, CN)],
                lsem.at[sl],
            )

        def cvt_copy(cc, sl):
            return pltpu.make_async_copy(
                cvo.at[sl],
                out_ref.at[:, pl.ds((1 - my_x) * N + cc * CN, CN)],
                lsem2.at[sl],
            )

        @pl.when((c == 0) & (k == 0))
        def _barrier():
            bar = pltpu.get_barrier_semaphore()
            for tgt in (y_tgt, x_tgt):
                pl.semaphore_signal(
                    bar, inc=1, device_id=tgt,
                    device_id_type=pl.DeviceIdType.MESH,
                )
            pl.semaphore_wait(bar, 2)

        @pl.when((k == 0) & (c >= 2))
        def _wait_prev_y():
            y_rdma(c - 2, slot).wait_send()

        @pl.when((k == 0) & (c >= 3))
        def _wait_prev_x():
            x_rdma(c - 3, 1 - slot).wait_send()
            out_copy(c - 3, 1 - slot).wait()

        @pl.when((k == 0) & (c >= 5))
        def _cvt_wait():
            cvt_copy(c - 5, 1 - slot).wait()

        @pl.when((k == 0) & (c >= 3))
        def _cvt():
            cc = c - 3
            sl = 1 - slot
            x_rdma(cc, sl).wait_recv()
            ld = pltpu.make_async_copy(
                stage2_ref.at[cc], cvb.at[sl], lsem2.at[sl]
            )
            ld.start()
            ld.wait()
            cvo[sl] = cvb[sl].astype(jnp.float32)
            cvt_copy(cc, sl).start()

        @pl.when(k == 0)
        def _zero():
            acc[slot] = jnp.zeros((Mx, CN), jnp.float32)

        bb = dy_ref[...]
        dn = (((0,), (0,)), ((), ()))
        acc[slot] += lax.dot_general(
            x_ref[...], bb, dn, preferred_element_type=jnp.float32
        )

        @pl.when(k == NK - 1)
        def _send_y():
            psb[slot] = acc[slot, pl.ds((1 - my_y) * H, H), :].astype(
                jnp.bfloat16
            )
            y_rdma(c, slot).start()

        def add_phase(cm1):
            sl = lax.rem(cm1, 2)
            y_rdma(cm1, sl).wait_recv()
            ld = pltpu.make_async_copy(
                stage_ref.at[cm1], rvb.at[sl], lsem.at[sl]
            )
            ld.start()
            ld.wait()
            ov = acc[sl, pl.ds(my_y * H, H), :] + rvb[sl].astype(jnp.float32)
            o[sl] = ov
            obf[sl] = ov.astype(jnp.bfloat16)
            out_copy(cm1, sl).start()
            x_rdma(cm1, sl).start()

        @pl.when((k == NK - 1) & (c >= 1))
        def _add_mid():
            add_phase(c - 1)

        @pl.when((c == NC - 1) & (k == NK - 1))
        def _final():
            sl_last = (NC - 1) % 2
            x_rdma(NC - 3, sl_last).wait_send()
            out_copy(NC - 3, sl_last).wait()
            add_phase(NC - 1)
            for cc in (NC - 3, NC - 2, NC - 1):
                sl = cc % 2
                cvt_copy(cc - 2, sl).wait()
                x_rdma(cc, sl).wait_recv()
                ld = pltpu.make_async_copy(
                    stage2_ref.at[cc], cvb.at[sl], lsem2.at[sl]
                )
                ld.start()
                ld.wait()
                cvo[sl] = cvb[sl].astype(jnp.float32)
                cvt_copy(cc, sl).start()
            for cc in (NC - 2, NC - 1):
                cvt_copy(cc, cc % 2).wait()
            for cc in (NC - 2, NC - 1):
                y_rdma(cc, cc % 2).wait_send()
                x_rdma(cc, cc % 2).wait_send()
                out_copy(cc, cc % 2).wait()

    grid_spec = pltpu.PrefetchScalarGridSpec(
        num_scalar_prefetch=1,
        grid=(NC, NK),
        in_specs=[
            pl.BlockSpec((BK, Mx), lambda c, k, s: (k, 0)),
            pl.BlockSpec((BK, CN), lambda c, k, s: (k, s[0] * NC + c)),
        ],
        out_specs=(
            pl.BlockSpec(memory_space=pltpu.MemorySpace.HBM),
            pl.BlockSpec(memory_space=pltpu.MemorySpace.HBM),
            pl.BlockSpec(memory_space=pltpu.MemorySpace.HBM),
        ),
        scratch_shapes=[
            pltpu.VMEM((2, Mx, CN), jnp.float32),
            pltpu.VMEM((2, H, CN), jnp.float32),
            pltpu.VMEM((2, H, CN), jnp.bfloat16),
            pltpu.VMEM((2, H, CN), jnp.bfloat16),
            pltpu.VMEM((2, H, CN), jnp.bfloat16),
            pltpu.VMEM((2, H, CN), jnp.bfloat16),
            pltpu.VMEM((2, H, CN), jnp.float32),
            pltpu.SemaphoreType.DMA((NC,)),
            pltpu.SemaphoreType.DMA((NC,)),
            pltpu.SemaphoreType.DMA((NC,)),
            pltpu.SemaphoreType.DMA((NC,)),
            pltpu.SemaphoreType.DMA((2,)),
            pltpu.SemaphoreType.DMA((2,)),
        ],
    )
    out, _, _ = pl.pallas_call(
        body,
        grid_spec=grid_spec,
        out_shape=(
            jax.ShapeDtypeStruct((H, 2 * N), jnp.float32),
            jax.ShapeDtypeStruct((NC, H, CN), jnp.bfloat16),
            jax.ShapeDtypeStruct((NC, H, CN), jnp.bfloat16),
        ),
        compiler_params=pltpu.CompilerParams(
            dimension_semantics=("arbitrary", "arbitrary"),
            collective_id=0,
            has_side_effects=True,
            vmem_limit_bytes=62 * 1024 * 1024,
        ),
    )(s, x, dy)
    return out
